# Initial kernel scaffold; baseline (speedup 1.0000x reference)
import jax
import jax.numpy as jnp
from jax.experimental import pallas as pl


def kernel(hidden_states, weight, e_score_correction_bias):
    raise NotImplementedError("write your pallas kernel here")


# speedup vs baseline: 1.1182x; 1.1182x over previous
import functools

import jax
import jax.numpy as jnp
from jax.experimental import pallas as pl

TOP_K = 8
N_EXPERTS = 64
N_GROUP = 8
GROUP_SIZE = N_EXPERTS // N_GROUP
TOPK_GROUP = 4
ROUTE_SCALE = 2.5
DIM = 2048
TOKENS = 16384

BLOCK_T = 512

_NEG_INF = float('-inf')


def _first_argmax(work, iota, width):
    m = jnp.max(work, axis=1, keepdims=True)
    idx = jnp.min(jnp.where(work == m, iota, width), axis=1, keepdims=True)
    return m, idx


def _router_body(x_ref, wt_ref, bias_ref, idx_ref, w_ref):
    x = x_ref[...]
    wt = wt_ref[...]
    logits = jax.lax.dot_general(
        x, wt, (((1,), (0,)), ((), ())), preferred_element_type=jnp.float32
    )
    scores = jax.nn.sigmoid(logits)
    s = scores + bias_ref[...]

    b = s.shape[0]
    iota_g = jax.lax.broadcasted_iota(jnp.int32, (b, GROUP_SIZE), 1)

    group_cols = []
    for g in range(N_GROUP):
        gs = s[:, g * GROUP_SIZE:(g + 1) * GROUP_SIZE]
        m1, i1 = _first_argmax(gs, iota_g, GROUP_SIZE)
        m2 = jnp.max(jnp.where(iota_g == i1, _NEG_INF, gs), axis=1, keepdims=True)
        group_cols.append(m1 + m2)
    group_scores = jnp.concatenate(group_cols, axis=1)

    iota_ng = jax.lax.broadcasted_iota(jnp.int32, (b, N_GROUP), 1)
    gwork = group_scores
    keep = jnp.zeros((b, N_GROUP), dtype=jnp.bool_)
    for _ in range(TOPK_GROUP):
        _, gi = _first_argmax(gwork, iota_ng, N_GROUP)
        sel = iota_ng == gi
        keep = jnp.logical_or(keep, sel)
        gwork = jnp.where(sel, _NEG_INF, gwork)

    s_masked = jnp.concatenate(
        [
            jnp.where(keep[:, g:g + 1], s[:, g * GROUP_SIZE:(g + 1) * GROUP_SIZE], 0.0)
            for g in range(N_GROUP)
        ],
        axis=1,
    )

    iota_e = jax.lax.broadcasted_iota(jnp.int32, (b, N_EXPERTS), 1)
    work = s_masked
    idx_cols = []
    w_cols = []
    for _ in range(TOP_K):
        _, ei = _first_argmax(work, iota_e, N_EXPERTS)
        selmask = iota_e == ei
        wk = jnp.max(jnp.where(selmask, scores, _NEG_INF), axis=1, keepdims=True)
        work = jnp.where(selmask, _NEG_INF, work)
        idx_cols.append(ei)
        w_cols.append(wk)

    topk_idx = jnp.concatenate(idx_cols, axis=1)
    topk_w = jnp.concatenate(w_cols, axis=1)
    denom = jnp.sum(topk_w, axis=1, keepdims=True) + 1e-20
    topk_w = topk_w / denom * ROUTE_SCALE

    idx_ref[...] = topk_idx
    w_ref[...] = topk_w


def kernel(hidden_states, weight, e_score_correction_bias, interpret=False):
    x = hidden_states.reshape(-1, DIM).astype(jnp.float32)
    wt = weight.astype(jnp.float32).T
    bias = e_score_correction_bias.astype(jnp.float32).reshape(1, N_EXPERTS)
    n_tokens = x.shape[0]
    grid = (n_tokens // BLOCK_T,)
    out_shapes = (
        jax.ShapeDtypeStruct((n_tokens, TOP_K), jnp.int32),
        jax.ShapeDtypeStruct((n_tokens, TOP_K), jnp.float32),
    )
    return pl.pallas_call(
        _router_body,
        grid=grid,
        in_specs=[
            pl.BlockSpec((BLOCK_T, DIM), lambda i: (i, 0)),
            pl.BlockSpec((DIM, N_EXPERTS), lambda i: (0, 0)),
            pl.BlockSpec((1, N_EXPERTS), lambda i: (0, 0)),
        ],
        out_specs=(
            pl.BlockSpec((BLOCK_T, TOP_K), lambda i: (i, 0)),
            pl.BlockSpec((BLOCK_T, TOP_K), lambda i: (i, 0)),
        ),
        out_shape=out_shapes,
        interpret=interpret,
    )(x, wt, bias)


# speedup vs baseline: 6.7965x; 6.0780x over previous
import jax
import jax.numpy as jnp
from jax.experimental import pallas as pl

TOP_K = 8
N_EXPERTS = 64
N_GROUP = 8
GROUP_SIZE = N_EXPERTS // N_GROUP
TOPK_GROUP = 4
ROUTE_SCALE = 2.5
DIM = 2048
TOKENS = 16384

BLOCK_T = 512

_NEG_INF = float('-inf')


def _first_argmax0(work, iota, width):
    m = jnp.max(work, axis=0, keepdims=True)
    idx = jnp.min(jnp.where(work == m, iota, width), axis=0, keepdims=True)
    return m, idx


def _router_body(x_ref, wt_ref, bias_ref, idx_ref, w_ref):
    x = x_ref[...]
    wt = wt_ref[...]
    logits = jax.lax.dot_general(
        x, wt, (((1,), (0,)), ((), ())), preferred_element_type=jnp.float32
    )
    lt = logits.T
    scores = jax.nn.sigmoid(lt)
    s = scores + bias_ref[...]

    b = s.shape[1]
    iota_g = jax.lax.broadcasted_iota(jnp.int32, (GROUP_SIZE, b), 0)

    group_rows = []
    for g in range(N_GROUP):
        gs = s[g * GROUP_SIZE:(g + 1) * GROUP_SIZE, :]
        m1, i1 = _first_argmax0(gs, iota_g, GROUP_SIZE)
        m2 = jnp.max(jnp.where(iota_g == i1, _NEG_INF, gs), axis=0, keepdims=True)
        group_rows.append(m1 + m2)
    group_scores = jnp.concatenate(group_rows, axis=0)

    iota_ng = jax.lax.broadcasted_iota(jnp.int32, (N_GROUP, b), 0)
    gwork = group_scores
    keep = jnp.zeros((N_GROUP, b), dtype=jnp.bool_)
    for _ in range(TOPK_GROUP):
        _, gi = _first_argmax0(gwork, iota_ng, N_GROUP)
        sel = iota_ng == gi
        keep = jnp.logical_or(keep, sel)
        gwork = jnp.where(sel, _NEG_INF, gwork)

    s_masked = jnp.concatenate(
        [
            jnp.where(keep[g:g + 1, :], s[g * GROUP_SIZE:(g + 1) * GROUP_SIZE, :], 0.0)
            for g in range(N_GROUP)
        ],
        axis=0,
    )

    iota_e = jax.lax.broadcasted_iota(jnp.int32, (N_EXPERTS, b), 0)
    work = s_masked
    idx_rows = []
    w_rows = []
    for _ in range(TOP_K):
        _, ei = _first_argmax0(work, iota_e, N_EXPERTS)
        selmask = iota_e == ei
        wk = jnp.max(jnp.where(selmask, scores, _NEG_INF), axis=0, keepdims=True)
        work = jnp.where(selmask, _NEG_INF, work)
        idx_rows.append(ei)
        w_rows.append(wk)

    topk_idx = jnp.concatenate(idx_rows, axis=0)
    topk_w = jnp.concatenate(w_rows, axis=0)
    denom = jnp.sum(topk_w, axis=0, keepdims=True) + 1e-20
    topk_w = topk_w / denom * ROUTE_SCALE

    idx_ref[...] = topk_idx
    w_ref[...] = topk_w


def kernel(hidden_states, weight, e_score_correction_bias, interpret=False):
    x = hidden_states.reshape(-1, DIM).astype(jnp.float32)
    wt = weight.astype(jnp.float32).T
    bias = e_score_correction_bias.astype(jnp.float32).reshape(N_EXPERTS, 1)
    n_tokens = x.shape[0]
    grid = (n_tokens // BLOCK_T,)
    out_shapes = (
        jax.ShapeDtypeStruct((TOP_K, n_tokens), jnp.int32),
        jax.ShapeDtypeStruct((TOP_K, n_tokens), jnp.float32),
    )
    idx_t, w_t = pl.pallas_call(
        _router_body,
        grid=grid,
        in_specs=[
            pl.BlockSpec((BLOCK_T, DIM), lambda i: (i, 0)),
            pl.BlockSpec((DIM, N_EXPERTS), lambda i: (0, 0)),
            pl.BlockSpec((N_EXPERTS, 1), lambda i: (0, 0)),
        ],
        out_specs=(
            pl.BlockSpec((TOP_K, BLOCK_T), lambda i: (0, i)),
            pl.BlockSpec((TOP_K, BLOCK_T), lambda i: (0, i)),
        ),
        out_shape=out_shapes,
        interpret=interpret,
    )(x, wt, bias)
    return idx_t.T, w_t.T


# speedup vs baseline: 7.8520x; 1.1553x over previous
import jax
import jax.numpy as jnp
from jax.experimental import pallas as pl

TOP_K = 8
N_EXPERTS = 64
N_GROUP = 8
GROUP_SIZE = N_EXPERTS // N_GROUP
TOPK_GROUP = 4
ROUTE_SCALE = 2.5
DIM = 2048
TOKENS = 16384

BLOCK_T = 1024

_NEG_INF = float('-inf')


def _first_argmax0(work, iota, width):
    m = jnp.max(work, axis=0, keepdims=True)
    idx = jnp.min(jnp.where(work == m, iota, width), axis=0, keepdims=True)
    return m, idx


def _router_body(x_ref, wt_ref, bias_ref, idx_ref, w_ref):
    x = x_ref[...]
    wt = wt_ref[...]
    logits = jax.lax.dot_general(
        x, wt, (((1,), (0,)), ((), ())), preferred_element_type=jnp.float32
    )
    lt = logits.T
    scores = jax.nn.sigmoid(lt)
    s = scores + bias_ref[...]

    b = s.shape[1]
    iota_g = jax.lax.broadcasted_iota(jnp.int32, (GROUP_SIZE, b), 0)

    group_rows = []
    for g in range(N_GROUP):
        gs = s[g * GROUP_SIZE:(g + 1) * GROUP_SIZE, :]
        m1, i1 = _first_argmax0(gs, iota_g, GROUP_SIZE)
        m2 = jnp.max(jnp.where(iota_g == i1, _NEG_INF, gs), axis=0, keepdims=True)
        group_rows.append(m1 + m2)
    group_scores = jnp.concatenate(group_rows, axis=0)

    iota_ng = jax.lax.broadcasted_iota(jnp.int32, (N_GROUP, b), 0)
    gwork = group_scores
    keep = jnp.zeros((N_GROUP, b), dtype=jnp.bool_)
    for _ in range(TOPK_GROUP):
        _, gi = _first_argmax0(gwork, iota_ng, N_GROUP)
        sel = iota_ng == gi
        keep = jnp.logical_or(keep, sel)
        gwork = jnp.where(sel, _NEG_INF, gwork)

    s_masked = jnp.concatenate(
        [
            jnp.where(keep[g:g + 1, :], s[g * GROUP_SIZE:(g + 1) * GROUP_SIZE, :], 0.0)
            for g in range(N_GROUP)
        ],
        axis=0,
    )

    iota_e = jax.lax.broadcasted_iota(jnp.int32, (N_EXPERTS, b), 0)
    work = s_masked
    idx_rows = []
    w_rows = []
    for _ in range(TOP_K):
        _, ei = _first_argmax0(work, iota_e, N_EXPERTS)
        selmask = iota_e == ei
        wk = jnp.max(jnp.where(selmask, scores, _NEG_INF), axis=0, keepdims=True)
        work = jnp.where(selmask, _NEG_INF, work)
        idx_rows.append(ei)
        w_rows.append(wk)

    topk_idx = jnp.concatenate(idx_rows, axis=0)
    topk_w = jnp.concatenate(w_rows, axis=0)
    denom = jnp.sum(topk_w, axis=0, keepdims=True) + 1e-20
    topk_w = topk_w / denom * ROUTE_SCALE

    idx_ref[...] = topk_idx
    w_ref[...] = topk_w


def kernel(hidden_states, weight, e_score_correction_bias, interpret=False):
    x = hidden_states.reshape(-1, DIM).astype(jnp.float32)
    wt = weight.astype(jnp.float32).T
    bias = e_score_correction_bias.astype(jnp.float32).reshape(N_EXPERTS, 1)
    n_tokens = x.shape[0]
    grid = (n_tokens // BLOCK_T,)
    out_shapes = (
        jax.ShapeDtypeStruct((TOP_K, n_tokens), jnp.int32),
        jax.ShapeDtypeStruct((TOP_K, n_tokens), jnp.float32),
    )
    idx_t, w_t = pl.pallas_call(
        _router_body,
        grid=grid,
        in_specs=[
            pl.BlockSpec((BLOCK_T, DIM), lambda i: (i, 0)),
            pl.BlockSpec((DIM, N_EXPERTS), lambda i: (0, 0)),
            pl.BlockSpec((N_EXPERTS, 1), lambda i: (0, 0)),
        ],
        out_specs=(
            pl.BlockSpec((TOP_K, BLOCK_T), lambda i: (0, i)),
            pl.BlockSpec((TOP_K, BLOCK_T), lambda i: (0, i)),
        ),
        out_shape=out_shapes,
        interpret=interpret,
    )(x, wt, bias)
    return idx_t.T, w_t.T
